# Initial kernel scaffold; baseline (speedup 1.0000x reference)
import jax
import jax.numpy as jnp
from jax.experimental import pallas as pl


def kernel(x, emb, W_ih, W_hh, b_h, W_out, b_out):
    raise NotImplementedError("write your pallas kernel here")


# speedup vs baseline: 13.0218x; 13.0218x over previous
import functools

import jax
import jax.numpy as jnp
from jax import lax
from jax.experimental import pallas as pl
from jax.experimental.pallas import tpu as pltpu
from jax.experimental.pallas import tpu_sc as plsc


def _sc_gather_rows(table, idx_flat):
    V, D = table.shape
    N = idx_flat.shape[0]
    info = plsc.get_sparse_core_info()
    NW = info.num_cores * info.num_subcores
    assert N % NW == 0, (N, NW)
    per_w = N // NW
    SUB = 128
    CH = 1280 if per_w % 1280 == 0 else SUB
    assert per_w % CH == 0 and CH % SUB == 0
    k = CH // SUB
    n_groups = per_w // CH
    mesh = plsc.VectorSubcoreMesh(core_axis_name="c", subcore_axis_name="s")

    @functools.partial(
        pl.kernel,
        mesh=mesh,
        out_type=jax.ShapeDtypeStruct((N, D), jnp.float32),
        compiler_params=pltpu.CompilerParams(use_tc_tiling_on_sc=False),
        scratch_types=[
            pltpu.VMEM((CH,), jnp.int32),
            pltpu.VMEM((CH, D), jnp.float32),
            pltpu.SemaphoreType.DMA,
        ],
    )
    def gather_kernel(table_hbm, idx_hbm, out_hbm, idx_v, rows_v, sem):
        wid = lax.axis_index("s") * info.num_cores + lax.axis_index("c")
        base = wid * per_w

        def group(g, carry):
            off = base + g * CH
            pltpu.sync_copy(idx_hbm.at[pl.ds(off, CH)], idx_v)
            copies = [
                pltpu.async_copy(
                    table_hbm.at[idx_v.at[pl.ds(j * SUB, SUB)]],
                    rows_v.at[pl.ds(j * SUB, SUB)],
                    sem,
                )
                for j in range(k)
            ]
            for c in copies:
                c.wait()
            pltpu.sync_copy(rows_v, out_hbm.at[pl.ds(off, CH)])
            return carry

        lax.fori_loop(0, n_groups, group, 0)

    return gather_kernel(table, idx_flat)


def _tc_rnn_decode(x_encT, W_ih, W_hh, b_h, W_out, b_out, bB=512, bL=8):
    L, B, E = x_encT.shape
    H = W_hh.shape[0]
    O = W_out.shape[1]
    assert B % bB == 0 and L % bL == 0

    def body(x_ref, wih_ref, whh_ref, bh_ref, wout_ref, bout_ref, y_ref, h_ref):
        @pl.when(pl.program_id(1) == 0)
        def _():
            h_ref[...] = jnp.zeros_like(h_ref)

        h = h_ref[...]
        wih = wih_ref[...]
        whh = whh_ref[...]
        bh = bh_ref[...]
        wout = wout_ref[...]
        bout = bout_ref[...]
        for t in range(bL):
            xt = x_ref[t]
            h = jnp.tanh(
                jnp.dot(xt, wih, preferred_element_type=jnp.float32)
                + jnp.dot(h, whh, preferred_element_type=jnp.float32)
                + bh
            )
            y_ref[:, t, :] = jnp.dot(h, wout, preferred_element_type=jnp.float32) + bout
        h_ref[...] = h

    return pl.pallas_call(
        body,
        grid=(B // bB, L // bL),
        in_specs=[
            pl.BlockSpec((bL, bB, E), lambda b, l: (l, b, 0)),
            pl.BlockSpec((E, H), lambda b, l: (0, 0)),
            pl.BlockSpec((H, H), lambda b, l: (0, 0)),
            pl.BlockSpec((1, H), lambda b, l: (0, 0)),
            pl.BlockSpec((H, O), lambda b, l: (0, 0)),
            pl.BlockSpec((1, O), lambda b, l: (0, 0)),
        ],
        out_specs=pl.BlockSpec((bB, bL, O), lambda b, l: (b, l, 0)),
        out_shape=jax.ShapeDtypeStruct((B, L, O), jnp.float32),
        scratch_shapes=[pltpu.VMEM((bB, H), jnp.float32)],
    )(x_encT, W_ih, W_hh, b_h.reshape(1, H), W_out, b_out.reshape(1, O))


def kernel(x, emb, W_ih, W_hh, b_h, W_out, b_out):
    B, L = x.shape
    E = emb.shape[1]
    idx_flat = jnp.swapaxes(x, 0, 1).reshape(B * L).astype(jnp.int32)
    x_encT = _sc_gather_rows(emb, idx_flat).reshape(L, B, E)
    return _tc_rnn_decode(x_encT, W_ih, W_hh, b_h, W_out, b_out)


# speedup vs baseline: 14.2970x; 1.0979x over previous
import functools

import jax
import jax.numpy as jnp
from jax import lax
from jax.experimental import pallas as pl
from jax.experimental.pallas import tpu as pltpu
from jax.experimental.pallas import tpu_sc as plsc

_BG = 256


def _sc_gather_packed(table, x2d):
    B, L = x2d.shape
    V, D = table.shape
    info = plsc.get_sparse_core_info()
    NW = info.num_cores * info.num_subcores
    BPW = B // NW
    TB = 8
    assert B % NW == 0 and L % TB == 0 and BPW % 16 == 0 and BPW <= 128
    NTB = L // TB
    NBLK = B // (4 * _BG)
    WPB = NW // NBLK
    WPG = _BG // BPW
    assert NW % NBLK == 0 and _BG % BPW == 0
    mesh = plsc.VectorSubcoreMesh(core_axis_name="c", subcore_axis_name="s")

    @functools.partial(
        pl.kernel,
        mesh=mesh,
        out_type=jax.ShapeDtypeStruct((L, B // 4, 4 * D), jnp.float32),
        compiler_params=pltpu.CompilerParams(
            use_tc_tiling_on_sc=False, needs_layout_passes=False
        ),
        scratch_types=[
            pltpu.VMEM((BPW, TB), jnp.int32),
            pltpu.VMEM((TB * BPW,), jnp.int32),
            pltpu.VMEM((TB, BPW, D), jnp.float32),
            pltpu.SemaphoreType.DMA,
        ],
    )
    def gather_kernel(table_hbm, x_hbm, out_hbm, xtile_v, idx_v, rows_v, sem):
        wid = lax.axis_index("s") * info.num_cores + lax.axis_index("c")
        b0 = wid * BPW
        blk = wid // WPB
        rem = wid % WPB
        cgrp = rem // WPG
        g0 = blk * _BG + (rem % WPG) * BPW
        lane = lax.iota(jnp.int32, 16)

        def tblock(tb, carry):
            pltpu.sync_copy(
                x_hbm.at[pl.ds(b0, BPW), pl.ds(tb * TB, TB)], xtile_v
            )
            for t in range(TB):
                col = jnp.full((16,), t, jnp.int32)
                for g in range(BPW // 16):
                    rows = lane + (g * 16)
                    idx_v[pl.ds(t * BPW + g * 16, 16)] = plsc.load_gather(
                        xtile_v, [rows, col]
                    )
            copies = [
                pltpu.async_copy(
                    table_hbm.at[idx_v.at[pl.ds(t * BPW, BPW)]],
                    rows_v.at[t],
                    sem,
                )
                for t in range(TB)
            ]
            for c in copies:
                c.wait()
            pltpu.sync_copy(
                rows_v,
                out_hbm.at[
                    pl.ds(tb * TB, TB), pl.ds(g0, BPW), pl.ds(cgrp * D, D)
                ],
            )
            return carry

        lax.fori_loop(0, NTB, tblock, 0)

    return gather_kernel(table, x2d)


def _tc_rnn_decode(x_encP, W_ih4, W_hh4, b_h4, W_out4, b_out4, B, bL=8):
    L, G, E4 = x_encP.shape
    H4 = W_hh4.shape[0]
    O = 64
    bG = _BG
    assert G % bG == 0 and L % bL == 0

    def body(x_ref, wih_ref, whh_ref, bh_ref, wout_ref, bout_ref, y_ref, h_ref):
        @pl.when(pl.program_id(1) == 0)
        def _():
            h_ref[...] = jnp.zeros_like(h_ref)

        h = h_ref[...]
        wih = wih_ref[...]
        whh = whh_ref[...]
        bh = bh_ref[...]
        wout = wout_ref[...]
        bout = bout_ref[...]
        for t in range(bL):
            xt = x_ref[t]
            h = jnp.tanh(
                jnp.dot(xt, wih, preferred_element_type=jnp.float32)
                + jnp.dot(h, whh, preferred_element_type=jnp.float32)
                + bh
            )
            y = jnp.dot(h, wout, preferred_element_type=jnp.float32) + bout
            for c in range(4):
                y_ref[c * bG:(c + 1) * bG, t, :] = y[:, c * O:(c + 1) * O]
        h_ref[...] = h

    return pl.pallas_call(
        body,
        grid=(G // bG, L // bL),
        in_specs=[
            pl.BlockSpec((bL, bG, E4), lambda b, l: (l, b, 0)),
            pl.BlockSpec((E4, H4), lambda b, l: (0, 0)),
            pl.BlockSpec((H4, H4), lambda b, l: (0, 0)),
            pl.BlockSpec((1, H4), lambda b, l: (0, 0)),
            pl.BlockSpec((H4, H4), lambda b, l: (0, 0)),
            pl.BlockSpec((1, H4), lambda b, l: (0, 0)),
        ],
        out_specs=pl.BlockSpec((4 * bG, bL, O), lambda b, l: (b, l, 0)),
        out_shape=jax.ShapeDtypeStruct((B, L, O), jnp.float32),
        scratch_shapes=[pltpu.VMEM((bG, H4), jnp.float32)],
        compiler_params=pltpu.CompilerParams(
            dimension_semantics=("parallel", "arbitrary")
        ),
    )(x_encP, W_ih4, W_hh4, b_h4, W_out4, b_out4)


def kernel(x, emb, W_ih, W_hh, b_h, W_out, b_out):
    B, L = x.shape
    H = W_hh.shape[0]
    O = W_out.shape[1]
    x_encP = _sc_gather_packed(emb, x.astype(jnp.int32))
    I4 = jnp.eye(4, dtype=jnp.float32)
    W_ih4 = jnp.kron(I4, W_ih)
    W_hh4 = jnp.kron(I4, W_hh)
    W_out4 = jnp.kron(I4, W_out)
    b_h4 = jnp.tile(b_h, 4).reshape(1, 4 * H)
    b_out4 = jnp.tile(b_out, 4).reshape(1, 4 * O)
    return _tc_rnn_decode(x_encP, W_ih4, W_hh4, b_h4, W_out4, b_out4, B)


# speedup vs baseline: 27.0847x; 1.8944x over previous
import functools

import jax
import jax.numpy as jnp
from jax import lax
from jax.experimental import pallas as pl
from jax.experimental.pallas import tpu as pltpu
from jax.experimental.pallas import tpu_sc as plsc

_BG = 512


def _sc_gather_packed(table, x2d):
    B, L = x2d.shape
    V, D = table.shape
    info = plsc.get_sparse_core_info()
    NW = info.num_cores * info.num_subcores
    BPW = B // NW
    TB = 8
    assert B % NW == 0 and L % TB == 0 and BPW % 16 == 0 and BPW <= 128
    NTB = L // TB
    NBLK = B // (4 * _BG)
    WPB = NW // NBLK
    WPG = _BG // BPW
    assert NW % NBLK == 0 and _BG % BPW == 0
    mesh = plsc.VectorSubcoreMesh(core_axis_name="c", subcore_axis_name="s")

    @functools.partial(
        pl.kernel,
        mesh=mesh,
        out_type=jax.ShapeDtypeStruct((L, B // 4, 4 * D), jnp.float32),
        compiler_params=pltpu.CompilerParams(
            use_tc_tiling_on_sc=False, needs_layout_passes=False
        ),
        scratch_types=[
            pltpu.VMEM((BPW, TB), jnp.int32),
            pltpu.VMEM((TB * BPW,), jnp.int32),
            pltpu.VMEM((TB, BPW, D), jnp.float32),
            pltpu.SemaphoreType.DMA,
        ],
    )
    def gather_kernel(table_hbm, x_hbm, out_hbm, xtile_v, idx_v, rows_v, sem):
        wid = lax.axis_index("s") * info.num_cores + lax.axis_index("c")
        b0 = wid * BPW
        blk = wid // WPB
        rem = wid % WPB
        cgrp = rem // WPG
        g0 = blk * _BG + (rem % WPG) * BPW
        lane = lax.iota(jnp.int32, 16)

        def tblock(tb, carry):
            pltpu.sync_copy(
                x_hbm.at[pl.ds(b0, BPW), pl.ds(tb * TB, TB)], xtile_v
            )
            for t in range(TB):
                col = jnp.full((16,), t, jnp.int32)
                for g in range(BPW // 16):
                    rows = lane + (g * 16)
                    idx_v[pl.ds(t * BPW + g * 16, 16)] = plsc.load_gather(
                        xtile_v, [rows, col]
                    )
            copies = [
                pltpu.async_copy(
                    table_hbm.at[idx_v.at[pl.ds(t * BPW, BPW)]],
                    rows_v.at[t],
                    sem,
                )
                for t in range(TB)
            ]
            for c in copies:
                c.wait()
            pltpu.sync_copy(
                rows_v,
                out_hbm.at[
                    pl.ds(tb * TB, TB), pl.ds(g0, BPW), pl.ds(cgrp * D, D)
                ],
            )
            return carry

        lax.fori_loop(0, NTB, tblock, 0)

    return gather_kernel(table, x2d)


def _tc_rnn_decode(
    x_encP, W_ih4T, W_hh4T, b_h4T, W_outST, b_outT, B, col0, y_prev=None, bL=8
):
    L, G, E4 = x_encP.shape
    H4 = W_hh4T.shape[0]
    O = W_outST.shape[1]
    bG = _BG
    assert G % bG == 0 and L % bL == 0

    def body(x_ref, wih_ref, whh_ref, bh_ref, wout_ref, bout_ref, *rest):
        y_ref, h_ref = rest[-2], rest[-1]

        @pl.when(pl.program_id(1) == 0)
        def _():
            h_ref[...] = jnp.zeros_like(h_ref)

        h = h_ref[...]
        wih = wih_ref[...]
        whh = whh_ref[...]
        bh = bh_ref[...]
        bout = bout_ref[...]
        for t in range(bL):
            xtT = x_ref[t].T
            h = jnp.tanh(
                jnp.dot(wih, xtT, preferred_element_type=jnp.float32)
                + jnp.dot(whh, h, preferred_element_type=jnp.float32)
                + bh
            )
            for c in range(4):
                y_ref[t * O:(t + 1) * O, c * bG:(c + 1) * bG] = (
                    jnp.dot(wout_ref[c], h, preferred_element_type=jnp.float32)
                    + bout
                )
        h_ref[...] = h

    in_specs = [
        pl.BlockSpec((bL, bG, E4), lambda b, l: (l, b, 0)),
        pl.BlockSpec((H4, E4), lambda b, l: (0, 0)),
        pl.BlockSpec((H4, H4), lambda b, l: (0, 0)),
        pl.BlockSpec((H4, 1), lambda b, l: (0, 0)),
        pl.BlockSpec((4, O, H4), lambda b, l: (0, 0, 0)),
        pl.BlockSpec((O, 1), lambda b, l: (0, 0)),
    ]
    args = [x_encP, W_ih4T, W_hh4T, b_h4T, W_outST, b_outT]
    aliases = {}
    if y_prev is not None:
        in_specs.append(pl.BlockSpec(memory_space=pltpu.MemorySpace.HBM))
        args.append(y_prev)
        aliases = {6: 0}
    return pl.pallas_call(
        body,
        grid=(G // bG, L // bL),
        in_specs=in_specs,
        out_specs=pl.BlockSpec((bL * O, 4 * bG), lambda b, l: (l, col0 + b)),
        out_shape=jax.ShapeDtypeStruct((L * O, B), jnp.float32),
        scratch_shapes=[pltpu.VMEM((H4, bG), jnp.float32)],
        input_output_aliases=aliases,
        compiler_params=pltpu.CompilerParams(
            dimension_semantics=("parallel", "arbitrary")
        ),
    )(*args)


def kernel(x, emb, W_ih, W_hh, b_h, W_out, b_out):
    B, L = x.shape
    H = W_hh.shape[0]
    O = W_out.shape[1]
    emb_lin = jax.lax.optimization_barrier(emb.reshape(-1))
    embL = emb_lin.reshape(emb.shape)
    xi = x.astype(jnp.int32)
    Bh = B // 2
    x_encP0 = _sc_gather_packed(embL, xi[:Bh])
    x_encP1 = _sc_gather_packed(embL, xi[Bh:])
    I4 = jnp.eye(4, dtype=jnp.float32)
    W_ih4T = jnp.kron(I4, W_ih.T)
    W_hh4T = jnp.kron(I4, W_hh.T)
    W_out4 = jnp.kron(I4, W_out)
    W_outST = jnp.stack([W_out4[:, c * O:(c + 1) * O].T for c in range(4)])
    b_h4T = jnp.tile(b_h, 4).reshape(4 * H, 1)
    b_outT = b_out.reshape(O, 1)
    yT = _tc_rnn_decode(
        x_encP0, W_ih4T, W_hh4T, b_h4T, W_outST, b_outT, B, col0=0
    )
    yT = _tc_rnn_decode(
        x_encP1, W_ih4T, W_hh4T, b_h4T, W_outST, b_outT, B, col0=1, y_prev=yT
    )
    return yT.T.reshape(B, L, O)
